# baseline (speedup 1.0000x reference)
import jax
import jax.numpy as jnp
from jax import lax
from jax.experimental import pallas as pl
from jax.experimental.pallas import tpu as pltpu
from jax.experimental.pallas import tpu_sc as plsc

N = 10000
NPAD = 10240
E = 160000
NC, NS = 2, 16
NW = NC * NS
EPT = E // NW
K = 125
NJ = EPT // K
RPT = NPAD // NS
BM = 512
GRID = NPAD // BM

import functools


@functools.lru_cache(maxsize=None)
def _mesh():
    return plsc.VectorSubcoreMesh(
        core_axis_name="c", subcore_axis_name="s",
        num_cores=NC, num_subcores=NS)


def _deg_body(dst2, ones_h, z128, out, ones_v, dstv, acc):
    c = lax.axis_index("c")
    s = lax.axis_index("s")
    w = c * NS + s
    pltpu.sync_copy(ones_h, ones_v)
    pltpu.sync_copy(dst2.at[pl.ds(w * NJ, NJ)], dstv)
    pltpu.sync_copy(z128.at[pl.ds(s * RPT, RPT)], acc.at[pl.ds(s * RPT, RPT)])
    plsc.subcore_barrier()

    def body(j, carry):
        pltpu.sync_copy(ones_v, acc.at[dstv.at[j]], add=True)
        return carry

    lax.fori_loop(0, NJ, body, 0)
    plsc.subcore_barrier()
    pltpu.sync_copy(acc.at[pl.ds(s * RPT, RPT)], out.at[c, pl.ds(s * RPT, RPT)])


@functools.lru_cache(maxsize=None)
def _deg_kernel():
    return pl.kernel(
        _deg_body,
        out_type=jax.ShapeDtypeStruct((NC, NPAD, 128), jnp.float32),
        mesh=_mesh(),
        scratch_types=[
            pltpu.VMEM((K, 128), jnp.float32),
            pltpu.VMEM((NJ, K), jnp.int32),
            pltpu.VMEM_SHARED((NPAD, 128), jnp.float32),
        ],
    )


def _deg_call(dst2, ones_st, z128):
    return _deg_kernel()(dst2, ones_st, z128)


def _make_scatter(nchunks, F):
    def body(*refs):
        tables = refs[:nchunks]
        src2, dst2, zf = refs[nchunks:nchunks + 3]
        outs = refs[nchunks + 3:2 * nchunks + 3]
        srcv, dstv, stage0, stage1, acc, sema, semb = refs[2 * nchunks + 3:]
        c = lax.axis_index("c")
        s = lax.axis_index("s")
        w = c * NS + s
        pltpu.sync_copy(src2.at[pl.ds(w * NJ, NJ)], srcv)
        pltpu.sync_copy(dst2.at[pl.ds(w * NJ, NJ)], dstv)
        for f in range(nchunks):
            table = tables[f]
            pltpu.sync_copy(zf.at[pl.ds(s * RPT, RPT)], acc.at[pl.ds(s * RPT, RPT)])
            plsc.subcore_barrier()

            pltpu.async_copy(table.at[srcv.at[0]], stage0, sema)

            def body_i(i, carry, table=table):
                j0 = 2 * i
                j1 = j0 + 1
                pltpu.async_copy(table.at[srcv.at[j1]], stage1, semb)
                pltpu.make_async_copy(table.at[srcv.at[0]], stage0, sema).wait()
                pltpu.sync_copy(stage0, acc.at[dstv.at[j0]], add=True)
                jn = jnp.where(j0 + 2 < NJ, j0 + 2, 0)
                pltpu.async_copy(table.at[srcv.at[jn]], stage0, sema)
                pltpu.make_async_copy(table.at[srcv.at[0]], stage1, semb).wait()
                pltpu.sync_copy(stage1, acc.at[dstv.at[j1]], add=True)
                return carry

            lax.fori_loop(0, NJ // 2, body_i, 0)
            pltpu.make_async_copy(table.at[srcv.at[0]], stage0, sema).wait()
            plsc.subcore_barrier()
            pltpu.sync_copy(acc.at[pl.ds(s * RPT, RPT)],
                            outs[f].at[c, pl.ds(s * RPT, RPT)])

    def call(*args):
        return pl.kernel(
            body,
            out_type=[jax.ShapeDtypeStruct((NC, NPAD, F), jnp.float32)] * nchunks,
            mesh=_mesh(),
            scratch_types=[
                pltpu.VMEM((NJ, K), jnp.int32),
                pltpu.VMEM((NJ, K), jnp.int32),
                pltpu.VMEM((K, F), jnp.float32),
                pltpu.VMEM((K, F), jnp.float32),
                pltpu.VMEM_SHARED((NPAD, F), jnp.float32),
                pltpu.SemaphoreType.DMA,
                pltpu.SemaphoreType.DMA,
            ],
        )(*args)

    return call


_scat512 = _make_scatter(4, 128)
_scat128 = _make_scatter(1, 128)


def _mm1_body(x_ref, w_ref, o0, o1, o2, o3):
    h = jnp.dot(x_ref[...], w_ref[...], precision=lax.Precision.DEFAULT,
                preferred_element_type=jnp.float32)
    for cc, o in enumerate((o0, o1, o2, o3)):
        o[...] = h[:, cc * 128:(cc + 1) * 128]


def _mm1_call(x, W1):
    return pl.pallas_call(
        _mm1_body,
        grid=(GRID,),
        in_specs=[
            pl.BlockSpec((BM, 784), lambda i: (i, 0)),
            pl.BlockSpec((784, 512), lambda i: (0, 0)),
        ],
        out_specs=[pl.BlockSpec((BM, 128), lambda i: (i, 0))] * 4,
        out_shape=[jax.ShapeDtypeStruct((NPAD, 128), jnp.float32)] * 4,
    )(x, W1)


def _scale_body(degt_ref, h0, h1, h2, h3, o0, o1, o2, o3, odinv):
    dinv = lax.rsqrt(degt_ref[0, :, 0:1] + degt_ref[1, :, 0:1] + 1.0)
    for h, o in zip((h0, h1, h2, h3), (o0, o1, o2, o3)):
        o[...] = h[...] * dinv
    odinv[...] = dinv


def _scale_call(degp, h):
    return pl.pallas_call(
        _scale_body,
        grid=(GRID,),
        in_specs=[pl.BlockSpec((NC, BM, 128), lambda i: (0, i, 0))]
        + [pl.BlockSpec((BM, 128), lambda i: (i, 0))] * 4,
        out_specs=[pl.BlockSpec((BM, 128), lambda i: (i, 0))] * 4
        + [pl.BlockSpec((BM, 1), lambda i: (i, 0))],
        out_shape=[jax.ShapeDtypeStruct((NPAD, 128), jnp.float32)] * 4
        + [jax.ShapeDtypeStruct((NPAD, 1), jnp.float32)],
    )(degp, *h)


def _mm2_body(p0, p1, p2, p3, g0, g1, g2, g3, dinv_ref, b_ref, w_ref,
              o0, o1, o2, o3):
    dinv = dinv_ref[...]
    cols = []
    for cc, (p, g) in enumerate(zip((p0, p1, p2, p3), (g0, g1, g2, g3))):
        t = p[0] + p[1] + g[...]
        cols.append(jnp.maximum(dinv * t + b_ref[cc, :], 0.0))
    a = jnp.concatenate(cols, axis=1)
    h = jnp.dot(a, w_ref[...], precision=lax.Precision.DEFAULT,
                preferred_element_type=jnp.float32)
    hd = h * dinv
    for cc, o in enumerate((o0, o1, o2, o3)):
        o[...] = hd[:, cc * 128:(cc + 1) * 128]


def _mm2_call(p, g, dinv, brow, W2):
    return pl.pallas_call(
        _mm2_body,
        grid=(GRID,),
        in_specs=(
            [pl.BlockSpec((NC, BM, 128), lambda i: (0, i, 0))] * 4
            + [pl.BlockSpec((BM, 128), lambda i: (i, 0))] * 4
            + [
                pl.BlockSpec((BM, 1), lambda i: (i, 0)),
                pl.BlockSpec((4, 128), lambda i: (0, 0)),
                pl.BlockSpec((512, 512), lambda i: (0, 0)),
            ]
        ),
        out_specs=[pl.BlockSpec((BM, 128), lambda i: (i, 0))] * 4,
        out_shape=[jax.ShapeDtypeStruct((NPAD, 128), jnp.float32)] * 4,
    )(*p, *g, dinv, brow, W2)


def _mm3_body(p0, p1, p2, p3, g0, g1, g2, g3, dinv_ref, b_ref, w_ref, o):
    dinv = dinv_ref[...]
    cols = []
    for cc, (p, g) in enumerate(zip((p0, p1, p2, p3), (g0, g1, g2, g3))):
        t = p[0] + p[1] + g[...]
        cols.append(jnp.maximum(dinv * t + b_ref[cc, :], 0.0))
    a = jnp.concatenate(cols, axis=1)
    h = jnp.dot(a, w_ref[...], precision=lax.Precision.DEFAULT,
                preferred_element_type=jnp.float32)
    o[...] = h * dinv


def _mm3_call(p, g, dinv, brow, Wcp):
    return pl.pallas_call(
        _mm3_body,
        grid=(GRID,),
        in_specs=(
            [pl.BlockSpec((NC, BM, 128), lambda i: (0, i, 0))] * 4
            + [pl.BlockSpec((BM, 128), lambda i: (i, 0))] * 4
            + [
                pl.BlockSpec((BM, 1), lambda i: (i, 0)),
                pl.BlockSpec((4, 128), lambda i: (0, 0)),
                pl.BlockSpec((512, 128), lambda i: (0, 0)),
            ]
        ),
        out_specs=pl.BlockSpec((BM, 128), lambda i: (i, 0)),
        out_shape=jax.ShapeDtypeStruct((NPAD, 128), jnp.float32),
    )(*p, *g, dinv, brow, Wcp)


def _fin_body(q_ref, g3_ref, dinv_ref, bc_ref, out_ref):
    t = q_ref[0] + q_ref[1] + g3_ref[...]
    logits = dinv_ref[...] * t + bc_ref[...]
    mask = lax.broadcasted_iota(jnp.int32, (BM, 128), 1) < 10
    lm = jnp.where(mask, logits, -1e30)
    m = jnp.max(lm, axis=1, keepdims=True)
    e = jnp.where(mask, jnp.exp(logits - m), 0.0)
    ssum = jnp.sum(e, axis=1, keepdims=True)
    res = (logits - m) - jnp.log(ssum)
    out_ref[...] = res[:, :16]


def _fin_call(q, g3, dinv, bcp):
    return pl.pallas_call(
        _fin_body,
        grid=(GRID,),
        in_specs=[
            pl.BlockSpec((NC, BM, 128), lambda i: (0, i, 0)),
            pl.BlockSpec((BM, 128), lambda i: (i, 0)),
            pl.BlockSpec((BM, 1), lambda i: (i, 0)),
            pl.BlockSpec((1, 128), lambda i: (0, 0)),
        ],
        out_specs=pl.BlockSpec((BM, 16), lambda i: (i, 0)),
        out_shape=jax.ShapeDtypeStruct((NPAD, 16), jnp.float32),
    )(q, g3, dinv, bcp)


def kernel(x, edge_index, W1, b1, W2, b2, Wc, bc):
    x = x.reshape(-1, 784)
    src = edge_index[0].astype(jnp.int32)
    dst = edge_index[1].astype(jnp.int32)
    src2 = src.reshape(NW * NJ, K)
    dst2 = dst.reshape(NW * NJ, K)
    z128 = jnp.zeros((NPAD, 128), jnp.float32)
    ones_st = jnp.ones((K, 128), jnp.float32)
    Wcp = jnp.pad(Wc, ((0, 0), (0, 118)))
    bcp = jnp.pad(bc, (0, 118)).reshape(1, 128)
    b1r = b1.reshape(4, 128)
    b2r = b2.reshape(4, 128)

    degp = _deg_call(dst2, ones_st, z128)
    h1 = _mm1_call(x, W1)
    *g1, dinv = _scale_call(degp, h1)
    p1 = _scat512(*g1, src2, dst2, z128)
    g2 = _mm2_call(p1, g1, dinv, b1r, W2)
    p2 = _scat512(*g2, src2, dst2, z128)
    g3 = _mm3_call(p2, g2, dinv, b2r, Wcp)
    q = _scat128(g3, src2, dst2, z128)
    out = _fin_call(q[0], g3, dinv, bcp)
    return out[:N, :10]


# speedup vs baseline: 1.0204x; 1.0204x over previous
import jax
import jax.numpy as jnp
from jax import lax
from jax.experimental import pallas as pl
from jax.experimental.pallas import tpu as pltpu
from jax.experimental.pallas import tpu_sc as plsc

N = 10000
NPAD = 10240
E = 160000
NC, NS = 2, 16
NW = NC * NS
EPT = E // NW
K = 125
NJ = EPT // K
RPT = NPAD // NS
BM = 512
GRID = NPAD // BM

import functools


@functools.lru_cache(maxsize=None)
def _mesh():
    return plsc.VectorSubcoreMesh(
        core_axis_name="c", subcore_axis_name="s",
        num_cores=NC, num_subcores=NS)


def _deg_body(dst2, ones_h, z128, out, ones_v, dstv, acc):
    c = lax.axis_index("c")
    s = lax.axis_index("s")
    w = c * NS + s
    pltpu.sync_copy(ones_h, ones_v)
    pltpu.sync_copy(dst2.at[pl.ds(w * NJ, NJ)], dstv)
    pltpu.sync_copy(z128.at[pl.ds(s * RPT, RPT)], acc.at[pl.ds(s * RPT, RPT)])
    plsc.subcore_barrier()

    def body(j, carry):
        pltpu.sync_copy(ones_v, acc.at[dstv.at[j]], add=True)
        return carry

    lax.fori_loop(0, NJ, body, 0)
    plsc.subcore_barrier()
    pltpu.sync_copy(acc.at[pl.ds(s * RPT, RPT)], out.at[c, pl.ds(s * RPT, RPT)])


@functools.lru_cache(maxsize=None)
def _deg_kernel():
    return pl.kernel(
        _deg_body,
        out_type=jax.ShapeDtypeStruct((NC, NPAD, 128), jnp.float32),
        mesh=_mesh(),
        scratch_types=[
            pltpu.VMEM((K, 128), jnp.float32),
            pltpu.VMEM((NJ, K), jnp.int32),
            pltpu.VMEM_SHARED((NPAD, 128), jnp.float32),
        ],
    )


def _deg_call(dst2, ones_st, z128):
    return _deg_kernel()(dst2, ones_st, z128)


def _make_scatter(nchunks, F):
    def body(*refs):
        tables = refs[:nchunks]
        src2, dst2, zf = refs[nchunks:nchunks + 3]
        outs = refs[nchunks + 3:2 * nchunks + 3]
        srcv, dstv, stage0, stage1, acc, sema, semb = refs[2 * nchunks + 3:]
        c = lax.axis_index("c")
        s = lax.axis_index("s")
        w = c * NS + s
        pltpu.sync_copy(src2.at[pl.ds(w * NJ, NJ)], srcv)
        pltpu.sync_copy(dst2.at[pl.ds(w * NJ, NJ)], dstv)
        for f in range(nchunks):
            table = tables[f]
            pltpu.sync_copy(zf.at[pl.ds(s * RPT, RPT)], acc.at[pl.ds(s * RPT, RPT)])
            plsc.subcore_barrier()

            pltpu.async_copy(table.at[srcv.at[0]], stage0, sema)

            def body_i(i, carry, table=table):
                j0 = 2 * i
                j1 = j0 + 1
                pltpu.async_copy(table.at[srcv.at[j1]], stage1, semb)
                pltpu.make_async_copy(table.at[srcv.at[0]], stage0, sema).wait()
                pltpu.sync_copy(stage0, acc.at[dstv.at[j0]], add=True)
                jn = jnp.where(j0 + 2 < NJ, j0 + 2, 0)
                pltpu.async_copy(table.at[srcv.at[jn]], stage0, sema)
                pltpu.make_async_copy(table.at[srcv.at[0]], stage1, semb).wait()
                pltpu.sync_copy(stage1, acc.at[dstv.at[j1]], add=True)
                return carry

            lax.fori_loop(0, NJ // 2, body_i, 0)
            pltpu.make_async_copy(table.at[srcv.at[0]], stage0, sema).wait()
            plsc.subcore_barrier()
            pltpu.sync_copy(acc.at[pl.ds(s * RPT, RPT)],
                            outs[f].at[c, pl.ds(s * RPT, RPT)])

    def call(*args):
        return pl.kernel(
            body,
            out_type=[jax.ShapeDtypeStruct((NC, NPAD, F), jnp.float32)] * nchunks,
            mesh=_mesh(),
            scratch_types=[
                pltpu.VMEM((NJ, K), jnp.int32),
                pltpu.VMEM((NJ, K), jnp.int32),
                pltpu.VMEM((K, F), jnp.float32),
                pltpu.VMEM((K, F), jnp.float32),
                pltpu.VMEM_SHARED((NPAD, F), jnp.float32),
                pltpu.SemaphoreType.DMA,
                pltpu.SemaphoreType.DMA,
            ],
        )(*args)

    return call


_scat512 = _make_scatter(4, 128)
_scat128 = _make_scatter(1, 128)


def _mm1_body(x_ref, w_ref, degp_ref, o0, o1, o2, o3, odinv):
    dinv = lax.rsqrt(degp_ref[0, :, 0:1] + degp_ref[1, :, 0:1] + 1.0)
    h = jnp.dot(x_ref[...], w_ref[...], precision=lax.Precision.DEFAULT,
                preferred_element_type=jnp.float32)
    hd = h * dinv
    for cc, o in enumerate((o0, o1, o2, o3)):
        o[...] = hd[:, cc * 128:(cc + 1) * 128]
    odinv[...] = dinv


def _mm1_call(x, W1, degp):
    return pl.pallas_call(
        _mm1_body,
        grid=(GRID,),
        in_specs=[
            pl.BlockSpec((BM, 784), lambda i: (i, 0)),
            pl.BlockSpec((784, 512), lambda i: (0, 0)),
            pl.BlockSpec((NC, BM, 128), lambda i: (0, i, 0)),
        ],
        out_specs=[pl.BlockSpec((BM, 128), lambda i: (i, 0))] * 4
        + [pl.BlockSpec((BM, 1), lambda i: (i, 0))],
        out_shape=[jax.ShapeDtypeStruct((NPAD, 128), jnp.float32)] * 4
        + [jax.ShapeDtypeStruct((NPAD, 1), jnp.float32)],
    )(x, W1, degp)


def _mm2_body(p0, p1, p2, p3, g0, g1, g2, g3, dinv_ref, b_ref, w_ref,
              o0, o1, o2, o3):
    dinv = dinv_ref[...]
    cols = []
    for cc, (p, g) in enumerate(zip((p0, p1, p2, p3), (g0, g1, g2, g3))):
        t = p[0] + p[1] + g[...]
        cols.append(jnp.maximum(dinv * t + b_ref[cc, :], 0.0))
    a = jnp.concatenate(cols, axis=1)
    h = jnp.dot(a, w_ref[...], precision=lax.Precision.DEFAULT,
                preferred_element_type=jnp.float32)
    hd = h * dinv
    for cc, o in enumerate((o0, o1, o2, o3)):
        o[...] = hd[:, cc * 128:(cc + 1) * 128]


def _mm2_call(p, g, dinv, brow, W2):
    return pl.pallas_call(
        _mm2_body,
        grid=(GRID,),
        in_specs=(
            [pl.BlockSpec((NC, BM, 128), lambda i: (0, i, 0))] * 4
            + [pl.BlockSpec((BM, 128), lambda i: (i, 0))] * 4
            + [
                pl.BlockSpec((BM, 1), lambda i: (i, 0)),
                pl.BlockSpec((4, 128), lambda i: (0, 0)),
                pl.BlockSpec((512, 512), lambda i: (0, 0)),
            ]
        ),
        out_specs=[pl.BlockSpec((BM, 128), lambda i: (i, 0))] * 4,
        out_shape=[jax.ShapeDtypeStruct((NPAD, 128), jnp.float32)] * 4,
    )(*p, *g, dinv, brow, W2)


def _mm3_body(p0, p1, p2, p3, g0, g1, g2, g3, dinv_ref, b_ref, w_ref, o):
    dinv = dinv_ref[...]
    cols = []
    for cc, (p, g) in enumerate(zip((p0, p1, p2, p3), (g0, g1, g2, g3))):
        t = p[0] + p[1] + g[...]
        cols.append(jnp.maximum(dinv * t + b_ref[cc, :], 0.0))
    a = jnp.concatenate(cols, axis=1)
    h = jnp.dot(a, w_ref[...], precision=lax.Precision.DEFAULT,
                preferred_element_type=jnp.float32)
    o[...] = h * dinv


def _mm3_call(p, g, dinv, brow, Wcp):
    return pl.pallas_call(
        _mm3_body,
        grid=(GRID,),
        in_specs=(
            [pl.BlockSpec((NC, BM, 128), lambda i: (0, i, 0))] * 4
            + [pl.BlockSpec((BM, 128), lambda i: (i, 0))] * 4
            + [
                pl.BlockSpec((BM, 1), lambda i: (i, 0)),
                pl.BlockSpec((4, 128), lambda i: (0, 0)),
                pl.BlockSpec((512, 128), lambda i: (0, 0)),
            ]
        ),
        out_specs=pl.BlockSpec((BM, 128), lambda i: (i, 0)),
        out_shape=jax.ShapeDtypeStruct((NPAD, 128), jnp.float32),
    )(*p, *g, dinv, brow, Wcp)


def _fin_body(q_ref, g3_ref, dinv_ref, bc_ref, out_ref):
    t = q_ref[0] + q_ref[1] + g3_ref[...]
    logits = dinv_ref[...] * t + bc_ref[...]
    mask = lax.broadcasted_iota(jnp.int32, (BM, 128), 1) < 10
    lm = jnp.where(mask, logits, -1e30)
    m = jnp.max(lm, axis=1, keepdims=True)
    e = jnp.where(mask, jnp.exp(logits - m), 0.0)
    ssum = jnp.sum(e, axis=1, keepdims=True)
    res = (logits - m) - jnp.log(ssum)
    out_ref[...] = res[:, :16]


def _fin_call(q, g3, dinv, bcp):
    return pl.pallas_call(
        _fin_body,
        grid=(GRID,),
        in_specs=[
            pl.BlockSpec((NC, BM, 128), lambda i: (0, i, 0)),
            pl.BlockSpec((BM, 128), lambda i: (i, 0)),
            pl.BlockSpec((BM, 1), lambda i: (i, 0)),
            pl.BlockSpec((1, 128), lambda i: (0, 0)),
        ],
        out_specs=pl.BlockSpec((BM, 16), lambda i: (i, 0)),
        out_shape=jax.ShapeDtypeStruct((NPAD, 16), jnp.float32),
    )(q, g3, dinv, bcp)


def kernel(x, edge_index, W1, b1, W2, b2, Wc, bc):
    x = x.reshape(-1, 784)
    src = edge_index[0].astype(jnp.int32)
    dst = edge_index[1].astype(jnp.int32)
    src2 = src.reshape(NW * NJ, K)
    dst2 = dst.reshape(NW * NJ, K)
    z128 = jnp.zeros((NPAD, 128), jnp.float32)
    ones_st = jnp.ones((K, 128), jnp.float32)
    Wcp = jnp.pad(Wc, ((0, 0), (0, 118)))
    bcp = jnp.pad(bc, (0, 118)).reshape(1, 128)
    b1r = b1.reshape(4, 128)
    b2r = b2.reshape(4, 128)

    degp = _deg_call(dst2, ones_st, z128)
    *g1, dinv = _mm1_call(x, W1, degp)
    p1 = _scat512(*g1, src2, dst2, z128)
    g2 = _mm2_call(p1, g1, dinv, b1r, W2)
    p2 = _scat512(*g2, src2, dst2, z128)
    g3 = _mm3_call(p2, g2, dinv, b2r, Wcp)
    q = _scat128(g3, src2, dst2, z128)
    out = _fin_call(q[0], g3, dinv, bcp)
    return out[:N, :10]


# speedup vs baseline: 1.0322x; 1.0115x over previous
import jax
import jax.numpy as jnp
from jax import lax
from jax.experimental import pallas as pl
from jax.experimental.pallas import tpu as pltpu
from jax.experimental.pallas import tpu_sc as plsc

N = 10000
NPAD = 10240
E = 160000
NC, NS = 2, 16
NW = NC * NS
EPT = E // NW
K = 125
NJ = EPT // K
RPT = NPAD // NS
BM = 512
GRID = NPAD // BM

import functools


@functools.lru_cache(maxsize=None)
def _mesh():
    return plsc.VectorSubcoreMesh(
        core_axis_name="c", subcore_axis_name="s",
        num_cores=NC, num_subcores=NS)


def _deg_body(dst2, ones_h, z128, out, ones_v, dstv, acc):
    c = lax.axis_index("c")
    s = lax.axis_index("s")
    w = c * NS + s
    pltpu.sync_copy(ones_h, ones_v)
    pltpu.sync_copy(dst2.at[pl.ds(w * NJ, NJ)], dstv)
    pltpu.sync_copy(z128.at[pl.ds(s * RPT, RPT)], acc.at[pl.ds(s * RPT, RPT)])
    plsc.subcore_barrier()

    def body(j, carry):
        pltpu.sync_copy(ones_v, acc.at[dstv.at[j]], add=True)
        return carry

    lax.fori_loop(0, NJ, body, 0)
    plsc.subcore_barrier()
    pltpu.sync_copy(acc.at[pl.ds(s * RPT, RPT)], out.at[c, pl.ds(s * RPT, RPT)])


@functools.lru_cache(maxsize=None)
def _deg_kernel():
    return pl.kernel(
        _deg_body,
        out_type=jax.ShapeDtypeStruct((NC, NPAD, 128), jnp.float32),
        mesh=_mesh(),
        scratch_types=[
            pltpu.VMEM((K, 128), jnp.float32),
            pltpu.VMEM((NJ, K), jnp.int32),
            pltpu.VMEM_SHARED((NPAD, 128), jnp.float32),
        ],
    )


def _deg_call(dst2, ones_st, z128):
    return _deg_kernel()(dst2, ones_st, z128)


def _make_scatter(nchunks, F):
    def body(*refs):
        tables = refs[:nchunks]
        src2, dst2, zf = refs[nchunks:nchunks + 3]
        outs = refs[nchunks + 3:2 * nchunks + 3]
        srcv, dstv, stage0, stage1, acc, sema, semb = refs[2 * nchunks + 3:]
        c = lax.axis_index("c")
        s = lax.axis_index("s")
        w = c * NS + s
        pltpu.sync_copy(src2.at[pl.ds(w * NJ, NJ)], srcv)
        pltpu.sync_copy(dst2.at[pl.ds(w * NJ, NJ)], dstv)
        for f in range(nchunks):
            table = tables[f]
            @pl.when(c == 0)
            def _seed(table=table):
                pltpu.sync_copy(table.at[pl.ds(s * RPT, RPT)],
                                acc.at[pl.ds(s * RPT, RPT)])

            @pl.when(c != 0)
            def _zero():
                pltpu.sync_copy(zf.at[pl.ds(s * RPT, RPT)],
                                acc.at[pl.ds(s * RPT, RPT)])

            plsc.subcore_barrier()

            pltpu.async_copy(table.at[srcv.at[0]], stage0, sema)

            def body_i(i, carry, table=table):
                j0 = 2 * i
                j1 = j0 + 1
                pltpu.async_copy(table.at[srcv.at[j1]], stage1, semb)
                pltpu.make_async_copy(table.at[srcv.at[0]], stage0, sema).wait()
                pltpu.sync_copy(stage0, acc.at[dstv.at[j0]], add=True)
                jn = jnp.where(j0 + 2 < NJ, j0 + 2, 0)
                pltpu.async_copy(table.at[srcv.at[jn]], stage0, sema)
                pltpu.make_async_copy(table.at[srcv.at[0]], stage1, semb).wait()
                pltpu.sync_copy(stage1, acc.at[dstv.at[j1]], add=True)
                return carry

            lax.fori_loop(0, NJ // 2, body_i, 0)
            pltpu.make_async_copy(table.at[srcv.at[0]], stage0, sema).wait()
            plsc.subcore_barrier()
            pltpu.sync_copy(acc.at[pl.ds(s * RPT, RPT)],
                            outs[f].at[c, pl.ds(s * RPT, RPT)])

    def call(*args):
        return pl.kernel(
            body,
            out_type=[jax.ShapeDtypeStruct((NC, NPAD, F), jnp.float32)] * nchunks,
            mesh=_mesh(),
            scratch_types=[
                pltpu.VMEM((NJ, K), jnp.int32),
                pltpu.VMEM((NJ, K), jnp.int32),
                pltpu.VMEM((K, F), jnp.float32),
                pltpu.VMEM((K, F), jnp.float32),
                pltpu.VMEM_SHARED((NPAD, F), jnp.float32),
                pltpu.SemaphoreType.DMA,
                pltpu.SemaphoreType.DMA,
            ],
        )(*args)

    return call


_scat512 = _make_scatter(4, 128)
_scat128 = _make_scatter(1, 128)


def _mm1_body(x_ref, w_ref, degp_ref, o0, o1, o2, o3, odinv):
    dinv = lax.rsqrt(degp_ref[0, :, 0:1] + degp_ref[1, :, 0:1] + 1.0)
    h = jnp.dot(x_ref[...], w_ref[...], precision=lax.Precision.DEFAULT,
                preferred_element_type=jnp.float32)
    hd = h * dinv
    for cc, o in enumerate((o0, o1, o2, o3)):
        o[...] = hd[:, cc * 128:(cc + 1) * 128]
    odinv[...] = dinv


def _mm1_call(x, W1, degp):
    return pl.pallas_call(
        _mm1_body,
        grid=(GRID,),
        in_specs=[
            pl.BlockSpec((BM, 784), lambda i: (i, 0)),
            pl.BlockSpec((784, 512), lambda i: (0, 0)),
            pl.BlockSpec((NC, BM, 128), lambda i: (0, i, 0)),
        ],
        out_specs=[pl.BlockSpec((BM, 128), lambda i: (i, 0))] * 4
        + [pl.BlockSpec((BM, 1), lambda i: (i, 0))],
        out_shape=[jax.ShapeDtypeStruct((NPAD, 128), jnp.float32)] * 4
        + [jax.ShapeDtypeStruct((NPAD, 1), jnp.float32)],
    )(x, W1, degp)


def _mm2_body(p0, p1, p2, p3, dinv_ref, b_ref, w_ref,
              o0, o1, o2, o3):
    dinv = dinv_ref[...]
    cols = []
    for cc, p in enumerate((p0, p1, p2, p3)):
        t = p[0] + p[1]
        cols.append(jnp.maximum(dinv * t + b_ref[cc, :], 0.0))
    a = jnp.concatenate(cols, axis=1)
    h = jnp.dot(a, w_ref[...], precision=lax.Precision.DEFAULT,
                preferred_element_type=jnp.float32)
    hd = h * dinv
    for cc, o in enumerate((o0, o1, o2, o3)):
        o[...] = hd[:, cc * 128:(cc + 1) * 128]


def _mm2_call(p, dinv, brow, W2):
    return pl.pallas_call(
        _mm2_body,
        grid=(GRID,),
        in_specs=(
            [pl.BlockSpec((NC, BM, 128), lambda i: (0, i, 0))] * 4
            + [
                pl.BlockSpec((BM, 1), lambda i: (i, 0)),
                pl.BlockSpec((4, 128), lambda i: (0, 0)),
                pl.BlockSpec((512, 512), lambda i: (0, 0)),
            ]
        ),
        out_specs=[pl.BlockSpec((BM, 128), lambda i: (i, 0))] * 4,
        out_shape=[jax.ShapeDtypeStruct((NPAD, 128), jnp.float32)] * 4,
    )(*p, dinv, brow, W2)


def _mm3_body(p0, p1, p2, p3, dinv_ref, b_ref, w_ref, o):
    dinv = dinv_ref[...]
    cols = []
    for cc, p in enumerate((p0, p1, p2, p3)):
        t = p[0] + p[1]
        cols.append(jnp.maximum(dinv * t + b_ref[cc, :], 0.0))
    a = jnp.concatenate(cols, axis=1)
    h = jnp.dot(a, w_ref[...], precision=lax.Precision.DEFAULT,
                preferred_element_type=jnp.float32)
    o[...] = h * dinv


def _mm3_call(p, dinv, brow, Wcp):
    return pl.pallas_call(
        _mm3_body,
        grid=(GRID,),
        in_specs=(
            [pl.BlockSpec((NC, BM, 128), lambda i: (0, i, 0))] * 4
            + [
                pl.BlockSpec((BM, 1), lambda i: (i, 0)),
                pl.BlockSpec((4, 128), lambda i: (0, 0)),
                pl.BlockSpec((512, 128), lambda i: (0, 0)),
            ]
        ),
        out_specs=pl.BlockSpec((BM, 128), lambda i: (i, 0)),
        out_shape=jax.ShapeDtypeStruct((NPAD, 128), jnp.float32),
    )(*p, dinv, brow, Wcp)


def _fin_body(q_ref, dinv_ref, bc_ref, out_ref):
    t = q_ref[0] + q_ref[1]
    logits = dinv_ref[...] * t + bc_ref[...]
    mask = lax.broadcasted_iota(jnp.int32, (BM, 128), 1) < 10
    lm = jnp.where(mask, logits, -1e30)
    m = jnp.max(lm, axis=1, keepdims=True)
    e = jnp.where(mask, jnp.exp(logits - m), 0.0)
    ssum = jnp.sum(e, axis=1, keepdims=True)
    res = (logits - m) - jnp.log(ssum)
    out_ref[...] = res[:, :16]


def _fin_call(q, dinv, bcp):
    return pl.pallas_call(
        _fin_body,
        grid=(GRID,),
        in_specs=[
            pl.BlockSpec((NC, BM, 128), lambda i: (0, i, 0)),
            pl.BlockSpec((BM, 1), lambda i: (i, 0)),
            pl.BlockSpec((1, 128), lambda i: (0, 0)),
        ],
        out_specs=pl.BlockSpec((BM, 16), lambda i: (i, 0)),
        out_shape=jax.ShapeDtypeStruct((NPAD, 16), jnp.float32),
    )(q, dinv, bcp)


def kernel(x, edge_index, W1, b1, W2, b2, Wc, bc):
    x = x.reshape(-1, 784)
    src = edge_index[0].astype(jnp.int32)
    dst = edge_index[1].astype(jnp.int32)
    src2 = src.reshape(NW * NJ, K)
    dst2 = dst.reshape(NW * NJ, K)
    z128 = jnp.zeros((NPAD, 128), jnp.float32)
    ones_st = jnp.ones((K, 128), jnp.float32)
    Wcp = jnp.pad(Wc, ((0, 0), (0, 118)))
    bcp = jnp.pad(bc, (0, 118)).reshape(1, 128)
    b1r = b1.reshape(4, 128)
    b2r = b2.reshape(4, 128)

    degp = _deg_call(dst2, ones_st, z128)
    *g1, dinv = _mm1_call(x, W1, degp)
    p1 = _scat512(*g1, src2, dst2, z128)
    g2 = _mm2_call(p1, dinv, b1r, W2)
    p2 = _scat512(*g2, src2, dst2, z128)
    g3 = _mm3_call(p2, dinv, b2r, Wcp)
    q = _scat128(g3, src2, dst2, z128)
    out = _fin_call(q[0], dinv, bcp)
    return out[:N, :10]


# speedup vs baseline: 1.0649x; 1.0316x over previous
import jax
import jax.numpy as jnp
from jax import lax
from jax.experimental import pallas as pl
from jax.experimental.pallas import tpu as pltpu
from jax.experimental.pallas import tpu_sc as plsc

N = 10000
NPAD = 10240
E = 160000
NC, NS = 2, 16
NW = NC * NS
EPT = E // NW
K = 125
NJ = EPT // K
RPT = NPAD // NS
BM = 1024
GRID = NPAD // BM

import functools


@functools.lru_cache(maxsize=None)
def _mesh():
    return plsc.VectorSubcoreMesh(
        core_axis_name="c", subcore_axis_name="s",
        num_cores=NC, num_subcores=NS)


def _deg_body(dst2, ones_h, z128, out, ones_v, dstv, acc):
    c = lax.axis_index("c")
    s = lax.axis_index("s")
    w = c * NS + s
    pltpu.sync_copy(ones_h, ones_v)
    pltpu.sync_copy(dst2.at[pl.ds(w * NJ, NJ)], dstv)
    pltpu.sync_copy(z128.at[pl.ds(s * RPT, RPT)], acc.at[pl.ds(s * RPT, RPT)])
    plsc.subcore_barrier()

    def body(j, carry):
        pltpu.sync_copy(ones_v, acc.at[dstv.at[j]], add=True)
        return carry

    lax.fori_loop(0, NJ, body, 0)
    plsc.subcore_barrier()
    pltpu.sync_copy(acc.at[pl.ds(s * RPT, RPT)], out.at[c, pl.ds(s * RPT, RPT)])


@functools.lru_cache(maxsize=None)
def _deg_kernel():
    return pl.kernel(
        _deg_body,
        out_type=jax.ShapeDtypeStruct((NC, NPAD, 128), jnp.float32),
        mesh=_mesh(),
        scratch_types=[
            pltpu.VMEM((K, 128), jnp.float32),
            pltpu.VMEM((NJ, K), jnp.int32),
            pltpu.VMEM_SHARED((NPAD, 128), jnp.float32),
        ],
    )


def _deg_call(dst2, ones_st, z128):
    return _deg_kernel()(dst2, ones_st, z128)


def _make_scatter(nchunks, F):
    def body(*refs):
        tables = refs[:nchunks]
        src2, dst2, zf = refs[nchunks:nchunks + 3]
        outs = refs[nchunks + 3:2 * nchunks + 3]
        srcv, dstv, stage0, stage1, acc, sema, semb = refs[2 * nchunks + 3:]
        c = lax.axis_index("c")
        s = lax.axis_index("s")
        w = c * NS + s
        pltpu.sync_copy(src2.at[pl.ds(w * NJ, NJ)], srcv)
        pltpu.sync_copy(dst2.at[pl.ds(w * NJ, NJ)], dstv)
        for f in range(nchunks):
            table = tables[f]
            @pl.when(c == 0)
            def _seed(table=table):
                pltpu.sync_copy(table.at[pl.ds(s * RPT, RPT)],
                                acc.at[pl.ds(s * RPT, RPT)])

            @pl.when(c != 0)
            def _zero():
                pltpu.sync_copy(zf.at[pl.ds(s * RPT, RPT)],
                                acc.at[pl.ds(s * RPT, RPT)])

            plsc.subcore_barrier()

            pltpu.async_copy(table.at[srcv.at[0]], stage0, sema)

            def body_i(i, carry, table=table):
                j0 = 2 * i
                j1 = j0 + 1
                pltpu.async_copy(table.at[srcv.at[j1]], stage1, semb)
                pltpu.make_async_copy(table.at[srcv.at[0]], stage0, sema).wait()
                pltpu.sync_copy(stage0, acc.at[dstv.at[j0]], add=True)
                jn = jnp.where(j0 + 2 < NJ, j0 + 2, 0)
                pltpu.async_copy(table.at[srcv.at[jn]], stage0, sema)
                pltpu.make_async_copy(table.at[srcv.at[0]], stage1, semb).wait()
                pltpu.sync_copy(stage1, acc.at[dstv.at[j1]], add=True)
                return carry

            lax.fori_loop(0, NJ // 2, body_i, 0)
            pltpu.make_async_copy(table.at[srcv.at[0]], stage0, sema).wait()
            plsc.subcore_barrier()
            pltpu.sync_copy(acc.at[pl.ds(s * RPT, RPT)],
                            outs[f].at[c, pl.ds(s * RPT, RPT)])

    def call(*args):
        return pl.kernel(
            body,
            out_type=[jax.ShapeDtypeStruct((NC, NPAD, F), jnp.float32)] * nchunks,
            mesh=_mesh(),
            scratch_types=[
                pltpu.VMEM((NJ, K), jnp.int32),
                pltpu.VMEM((NJ, K), jnp.int32),
                pltpu.VMEM((K, F), jnp.float32),
                pltpu.VMEM((K, F), jnp.float32),
                pltpu.VMEM_SHARED((NPAD, F), jnp.float32),
                pltpu.SemaphoreType.DMA,
                pltpu.SemaphoreType.DMA,
            ],
        )(*args)

    return call


_scat512 = _make_scatter(4, 128)
_scat128 = _make_scatter(1, 128)


def _mm1_body(x_ref, w_ref, degp_ref, o0, o1, o2, o3, odinv):
    dinv = lax.rsqrt(degp_ref[0, :, 0:1] + degp_ref[1, :, 0:1] + 1.0)
    h = jnp.dot(x_ref[...], w_ref[...], precision=lax.Precision.DEFAULT,
                preferred_element_type=jnp.float32)
    hd = h * dinv
    for cc, o in enumerate((o0, o1, o2, o3)):
        o[...] = hd[:, cc * 128:(cc + 1) * 128]
    odinv[...] = dinv


def _mm1_call(x, W1, degp):
    return pl.pallas_call(
        _mm1_body,
        grid=(GRID,),
        in_specs=[
            pl.BlockSpec((BM, 784), lambda i: (i, 0)),
            pl.BlockSpec((784, 512), lambda i: (0, 0)),
            pl.BlockSpec((NC, BM, 128), lambda i: (0, i, 0)),
        ],
        out_specs=[pl.BlockSpec((BM, 128), lambda i: (i, 0))] * 4
        + [pl.BlockSpec((BM, 1), lambda i: (i, 0))],
        out_shape=[jax.ShapeDtypeStruct((NPAD, 128), jnp.float32)] * 4
        + [jax.ShapeDtypeStruct((NPAD, 1), jnp.float32)],
    )(x, W1, degp)


def _mm2_body(p0, p1, p2, p3, dinv_ref, b_ref, w_ref,
              o0, o1, o2, o3):
    dinv = dinv_ref[...]
    cols = []
    for cc, p in enumerate((p0, p1, p2, p3)):
        t = p[0] + p[1]
        cols.append(jnp.maximum(dinv * t + b_ref[cc, :], 0.0))
    a = jnp.concatenate(cols, axis=1)
    h = jnp.dot(a, w_ref[...], precision=lax.Precision.DEFAULT,
                preferred_element_type=jnp.float32)
    hd = h * dinv
    for cc, o in enumerate((o0, o1, o2, o3)):
        o[...] = hd[:, cc * 128:(cc + 1) * 128]


def _mm2_call(p, dinv, brow, W2):
    return pl.pallas_call(
        _mm2_body,
        grid=(GRID,),
        in_specs=(
            [pl.BlockSpec((NC, BM, 128), lambda i: (0, i, 0))] * 4
            + [
                pl.BlockSpec((BM, 1), lambda i: (i, 0)),
                pl.BlockSpec((4, 128), lambda i: (0, 0)),
                pl.BlockSpec((512, 512), lambda i: (0, 0)),
            ]
        ),
        out_specs=[pl.BlockSpec((BM, 128), lambda i: (i, 0))] * 4,
        out_shape=[jax.ShapeDtypeStruct((NPAD, 128), jnp.float32)] * 4,
    )(*p, dinv, brow, W2)


def _mm3_body(p0, p1, p2, p3, dinv_ref, b_ref, w_ref, o):
    dinv = dinv_ref[...]
    cols = []
    for cc, p in enumerate((p0, p1, p2, p3)):
        t = p[0] + p[1]
        cols.append(jnp.maximum(dinv * t + b_ref[cc, :], 0.0))
    a = jnp.concatenate(cols, axis=1)
    h = jnp.dot(a, w_ref[...], precision=lax.Precision.DEFAULT,
                preferred_element_type=jnp.float32)
    o[...] = h * dinv


def _mm3_call(p, dinv, brow, Wcp):
    return pl.pallas_call(
        _mm3_body,
        grid=(GRID,),
        in_specs=(
            [pl.BlockSpec((NC, BM, 128), lambda i: (0, i, 0))] * 4
            + [
                pl.BlockSpec((BM, 1), lambda i: (i, 0)),
                pl.BlockSpec((4, 128), lambda i: (0, 0)),
                pl.BlockSpec((512, 128), lambda i: (0, 0)),
            ]
        ),
        out_specs=pl.BlockSpec((BM, 128), lambda i: (i, 0)),
        out_shape=jax.ShapeDtypeStruct((NPAD, 128), jnp.float32),
    )(*p, dinv, brow, Wcp)


def _fin_body(q_ref, dinv_ref, bc_ref, out_ref):
    t = q_ref[0] + q_ref[1]
    logits = dinv_ref[...] * t + bc_ref[...]
    mask = lax.broadcasted_iota(jnp.int32, (BM, 128), 1) < 10
    lm = jnp.where(mask, logits, -1e30)
    m = jnp.max(lm, axis=1, keepdims=True)
    e = jnp.where(mask, jnp.exp(logits - m), 0.0)
    ssum = jnp.sum(e, axis=1, keepdims=True)
    res = (logits - m) - jnp.log(ssum)
    out_ref[...] = res[:, :16]


def _fin_call(q, dinv, bcp):
    return pl.pallas_call(
        _fin_body,
        grid=(GRID,),
        in_specs=[
            pl.BlockSpec((NC, BM, 128), lambda i: (0, i, 0)),
            pl.BlockSpec((BM, 1), lambda i: (i, 0)),
            pl.BlockSpec((1, 128), lambda i: (0, 0)),
        ],
        out_specs=pl.BlockSpec((BM, 16), lambda i: (i, 0)),
        out_shape=jax.ShapeDtypeStruct((NPAD, 16), jnp.float32),
    )(q, dinv, bcp)


def kernel(x, edge_index, W1, b1, W2, b2, Wc, bc):
    x = x.reshape(-1, 784)
    src = edge_index[0].astype(jnp.int32)
    dst = edge_index[1].astype(jnp.int32)
    src2 = src.reshape(NW * NJ, K)
    dst2 = dst.reshape(NW * NJ, K)
    z128 = jnp.zeros((NPAD, 128), jnp.float32)
    ones_st = jnp.ones((K, 128), jnp.float32)
    Wcp = jnp.pad(Wc, ((0, 0), (0, 118)))
    bcp = jnp.pad(bc, (0, 118)).reshape(1, 128)
    b1r = b1.reshape(4, 128)
    b2r = b2.reshape(4, 128)

    degp = _deg_call(dst2, ones_st, z128)
    *g1, dinv = _mm1_call(x, W1, degp)
    p1 = _scat512(*g1, src2, dst2, z128)
    g2 = _mm2_call(p1, dinv, b1r, W2)
    p2 = _scat512(*g2, src2, dst2, z128)
    g3 = _mm3_call(p2, dinv, b2r, Wcp)
    q = _scat128(g3, src2, dst2, z128)
    out = _fin_call(q[0], dinv, bcp)
    return out[:N, :10]
